# baseline (speedup 1.0000x reference)
import jax
import jax.numpy as jnp
from jax.experimental import pallas as pl
from jax.experimental.pallas import tpu as pltpu

_BN_EPS = 1e-5


def _fused_call(xsc, w1t, w2f, gamma, beta, mean, var, tb):
    B, S, C = xsc.shape
    Cb = w1t.shape[0]
    Co = w2f.shape[1]
    nb = B // tb
    inv_spatial = 1.0 / float(S)

    flat_mask = (Co % 128 == 0) and ((tb * Co) // 128) % 8 == 0

    flat_norm = (B <= 128) and (nb >= 1)

    def body(x_ref, w1_ref, w2_ref, g_ref, b_ref, m_ref, v_ref,
             mask_ref, norm_ref, *scratch):
        bi = pl.program_id(0)
        part = x_ref[...].astype(jnp.float32)
        s = S
        while s > 8 and s % 2 == 0:
            half = s // 2
            part = part[:, :half, :] + part[:, half:s, :]
            s = half
        ctx = jnp.sum(part, axis=1) * inv_spatial
        h = jax.lax.dot_general(ctx, w1_ref[...],
                                (((1,), (1,)), ((), ())),
                                preferred_element_type=jnp.float32)
        inv_std = jax.lax.rsqrt(v_ref[...] + _BN_EPS)
        scale = g_ref[...] * inv_std
        shift = b_ref[...] - m_ref[...] * scale
        h = jnp.maximum(h * scale + shift, 0.0)
        logits = jnp.dot(h, w2_ref[...],
                         preferred_element_type=jnp.float32)
        mask = (logits >= 0.0).astype(jnp.float32)
        if flat_mask:
            mask_ref[...] = mask.reshape(mask_ref.shape)
        else:
            mask_ref[...] = mask
        if flat_norm:
            acc_ref = scratch[0]
            acc_ref[pl.ds(bi * tb, tb), :] = mask

            @pl.when(bi == nb - 1)
            def _():
                ones_row = jnp.ones((1, Co), jnp.float32)
                norms = jax.lax.dot_general(
                    ones_row, acc_ref[...], (((1,), (1,)), ((), ())),
                    preferred_element_type=jnp.float32)
                norm_ref[...] = norms[0]
        else:
            norm_ref[...] = jnp.sum(mask, axis=1, keepdims=True)

    rep = lambda bi: (0, 0)
    if flat_mask:
        mask_shape = ((B * Co) // 128, 128)
        mask_spec = pl.BlockSpec(((tb * Co) // 128, 128), lambda bi: (bi, 0))
    else:
        mask_shape = (B, Co)
        mask_spec = pl.BlockSpec((tb, Co), lambda bi: (bi, 0))
    if flat_norm:
        norm_shape = (B,)
        norm_spec = pl.BlockSpec((B,), lambda bi: (0,))
        scratch_shapes = [pltpu.VMEM((B, Co), jnp.float32)]
    else:
        norm_shape = (B, 1)
        norm_spec = pl.BlockSpec((tb, 1), lambda bi: (bi, 0))
        scratch_shapes = []
    return pl.pallas_call(
        body,
        out_shape=(jax.ShapeDtypeStruct(mask_shape, jnp.float32),
                   jax.ShapeDtypeStruct(norm_shape, jnp.float32)),
        grid_spec=pltpu.PrefetchScalarGridSpec(
            num_scalar_prefetch=0,
            grid=(nb,),
            in_specs=[pl.BlockSpec((tb, S, C), lambda bi: (bi, 0, 0)),
                      pl.BlockSpec((Cb, C), rep),
                      pl.BlockSpec((Cb, Co), rep),
                      pl.BlockSpec((1, Cb), rep),
                      pl.BlockSpec((1, Cb), rep),
                      pl.BlockSpec((1, Cb), rep),
                      pl.BlockSpec((1, Cb), rep)],
            out_specs=(mask_spec, norm_spec),
            scratch_shapes=scratch_shapes,
        ),
        compiler_params=pltpu.CompilerParams(
            dimension_semantics=("arbitrary",),
            vmem_limit_bytes=56 * 1024 * 1024),
    )(xsc, w1t, w2f, gamma, beta, mean, var)


def kernel(x, w1, bn_gamma, bn_beta, bn_mean, bn_var, w2):
    B, C, H, W = x.shape
    S = H * W
    Cb = w1.shape[1]
    Co = w2.shape[1]

    xsc = jnp.transpose(x, (0, 2, 3, 1)).reshape(B, S, C)
    w1t = jnp.transpose(w1)

    tb = B
    for cand in (16, 8):
        if B % cand == 0 and cand * S * C * x.dtype.itemsize <= 16 * 1024 * 1024:
            tb = cand
            break
    while tb > 1 and tb * S * C * x.dtype.itemsize > 16 * 1024 * 1024 \
            and tb % 2 == 0:
        tb //= 2

    f32 = jnp.float32
    mask2d, norm2d = _fused_call(
        xsc, w1t.astype(f32), w2.astype(f32),
        bn_gamma.astype(f32).reshape(1, Cb),
        bn_beta.astype(f32).reshape(1, Cb),
        bn_mean.astype(f32).reshape(1, Cb),
        bn_var.astype(f32).reshape(1, Cb),
        tb)

    mask_c = mask2d.reshape(B, Co, 1, 1)
    norm = norm2d.reshape(B)
    norm_t = jnp.array([Co], dtype=f32)
    return mask_c, norm, norm_t


# speedup vs baseline: 1.0830x; 1.0830x over previous
import jax
import jax.numpy as jnp
from jax.experimental import pallas as pl
from jax.experimental.pallas import tpu as pltpu

_BN_EPS = 1e-5


def _fused_call(xsc, w1t, w2f, gamma, beta, mean, var, tb):
    B, S, C = xsc.shape
    Cb = w1t.shape[0]
    Co = w2f.shape[1]
    nb = B // tb
    inv_spatial = 1.0 / float(S)

    flat_mask = (Co % 128 == 0) and ((tb * Co) // 128) % 8 == 0

    flat_norm = (B <= 128) and (nb >= 1)

    two_stream = (S % 2 == 0)
    S2 = S // 2 if two_stream else S

    def body(x_ref, x2_ref, w1_ref, w2_ref, g_ref, b_ref, m_ref, v_ref,
             mask_ref, norm_ref, *scratch):
        bi = pl.program_id(0)
        if two_stream:
            part = (x_ref[...].astype(jnp.float32)
                    + x2_ref[...].astype(jnp.float32))
        else:
            part = x_ref[...].astype(jnp.float32)
        s = S2
        while s > 8 and s % 2 == 0:
            half = s // 2
            part = part[:, :half, :] + part[:, half:s, :]
            s = half
        ctx = jnp.sum(part, axis=1) * inv_spatial
        h = jax.lax.dot_general(ctx, w1_ref[...],
                                (((1,), (1,)), ((), ())),
                                preferred_element_type=jnp.float32)
        inv_std = jax.lax.rsqrt(v_ref[...] + _BN_EPS)
        scale = g_ref[...] * inv_std
        shift = b_ref[...] - m_ref[...] * scale
        h = jnp.maximum(h * scale + shift, 0.0)
        logits = jnp.dot(h, w2_ref[...],
                         preferred_element_type=jnp.float32)
        mask = (logits >= 0.0).astype(jnp.float32)
        if flat_mask:
            mask_ref[...] = mask.reshape(mask_ref.shape)
        else:
            mask_ref[...] = mask
        if flat_norm:
            acc_ref = scratch[0]
            acc_ref[pl.ds(bi * tb, tb), :] = mask

            @pl.when(bi == nb - 1)
            def _():
                ones_row = jnp.ones((1, Co), jnp.float32)
                norms = jax.lax.dot_general(
                    ones_row, acc_ref[...], (((1,), (1,)), ((), ())),
                    preferred_element_type=jnp.float32)
                norm_ref[...] = norms[0]
        else:
            norm_ref[...] = jnp.sum(mask, axis=1, keepdims=True)

    rep = lambda bi: (0, 0)
    if flat_mask:
        mask_shape = ((B * Co) // 128, 128)
        mask_spec = pl.BlockSpec(((tb * Co) // 128, 128), lambda bi: (bi, 0))
    else:
        mask_shape = (B, Co)
        mask_spec = pl.BlockSpec((tb, Co), lambda bi: (bi, 0))
    if flat_norm:
        norm_shape = (B,)
        norm_spec = pl.BlockSpec((B,), lambda bi: (0,))
        scratch_shapes = [pltpu.VMEM((B, Co), jnp.float32)]
    else:
        norm_shape = (B, 1)
        norm_spec = pl.BlockSpec((tb, 1), lambda bi: (bi, 0))
        scratch_shapes = []
    return pl.pallas_call(
        body,
        out_shape=(jax.ShapeDtypeStruct(mask_shape, jnp.float32),
                   jax.ShapeDtypeStruct(norm_shape, jnp.float32)),
        grid_spec=pltpu.PrefetchScalarGridSpec(
            num_scalar_prefetch=0,
            grid=(nb,),
            in_specs=[pl.BlockSpec((tb, S2, C), lambda bi: (bi, 0, 0)),
                      pl.BlockSpec((tb, S2, C),
                                   (lambda bi: (bi, 1, 0)) if two_stream
                                   else (lambda bi: (bi, 0, 0))),
                      pl.BlockSpec((Cb, C), rep),
                      pl.BlockSpec((Cb, Co), rep),
                      pl.BlockSpec((1, Cb), rep),
                      pl.BlockSpec((1, Cb), rep),
                      pl.BlockSpec((1, Cb), rep),
                      pl.BlockSpec((1, Cb), rep)],
            out_specs=(mask_spec, norm_spec),
            scratch_shapes=scratch_shapes,
        ),
        compiler_params=pltpu.CompilerParams(
            dimension_semantics=("arbitrary",),
            vmem_limit_bytes=56 * 1024 * 1024),
    )(xsc, xsc, w1t, w2f, gamma, beta, mean, var)


def kernel(x, w1, bn_gamma, bn_beta, bn_mean, bn_var, w2):
    B, C, H, W = x.shape
    S = H * W
    Cb = w1.shape[1]
    Co = w2.shape[1]

    xsc = jnp.transpose(x, (0, 2, 3, 1)).reshape(B, S, C)
    w1t = jnp.transpose(w1)

    tb = B
    for cand in (32, 16, 8):
        if B % cand == 0 and cand * S * C * x.dtype.itemsize <= 8 * 1024 * 1024:
            tb = cand
            break
    while tb > 1 and tb * S * C * x.dtype.itemsize > 16 * 1024 * 1024 \
            and tb % 2 == 0:
        tb //= 2

    f32 = jnp.float32
    mask2d, norm2d = _fused_call(
        xsc, w1t.astype(f32), w2.astype(f32),
        bn_gamma.astype(f32).reshape(1, Cb),
        bn_beta.astype(f32).reshape(1, Cb),
        bn_mean.astype(f32).reshape(1, Cb),
        bn_var.astype(f32).reshape(1, Cb),
        tb)

    mask_c = mask2d.reshape(B, Co, 1, 1)
    norm = norm2d.reshape(B)
    norm_t = jnp.array([Co], dtype=f32)
    return mask_c, norm, norm_t


# speedup vs baseline: 1.0830x; 1.0000x over previous
import jax
import jax.numpy as jnp
from jax.experimental import pallas as pl
from jax.experimental.pallas import tpu as pltpu

_BN_EPS = 1e-5


def _fused_call(xsc, w1t, w2f, gamma, beta, mean, var, tb):
    B, S, C = xsc.shape
    Cb = w1t.shape[0]
    Co = w2f.shape[1]
    nb = B // tb
    inv_spatial = 1.0 / float(S)

    flat_mask = (Co % 128 == 0) and ((tb * Co) // 128) % 8 == 0

    flat_norm = (B <= 128) and (nb >= 1)

    def body(x_ref, w1_ref, w2_ref, g_ref, b_ref, m_ref, v_ref,
             mask_ref, norm_ref, *scratch):
        bi = pl.program_id(0)
        part = x_ref[...].astype(jnp.float32)
        s = S
        while s > 8 and s % 2 == 0:
            half = s // 2
            part = part[:, :half, :] + part[:, half:s, :]
            s = half
        ctx = jnp.sum(part, axis=1) * inv_spatial
        h = jax.lax.dot_general(ctx, w1_ref[...],
                                (((1,), (1,)), ((), ())),
                                preferred_element_type=jnp.float32)
        inv_std = jax.lax.rsqrt(v_ref[...] + _BN_EPS)
        scale = g_ref[...] * inv_std
        shift = b_ref[...] - m_ref[...] * scale
        h = jnp.maximum(h * scale + shift, 0.0)
        logits = jnp.dot(h, w2_ref[...],
                         preferred_element_type=jnp.float32)
        mask = (logits >= 0.0).astype(jnp.float32)
        if flat_mask:
            mask_ref[...] = mask.reshape(mask_ref.shape)
        else:
            mask_ref[...] = mask
        if flat_norm:
            acc_ref = scratch[0]
            acc_ref[pl.ds(bi * tb, tb), :] = mask

            @pl.when(bi == nb - 1)
            def _():
                ones_row = jnp.ones((1, Co), jnp.float32)
                norms = jax.lax.dot_general(
                    ones_row, acc_ref[...], (((1,), (1,)), ((), ())),
                    preferred_element_type=jnp.float32)
                norm_ref[...] = norms[0]
        else:
            norm_ref[...] = jnp.sum(mask, axis=1, keepdims=True)

    rep = lambda bi: (0, 0)
    if flat_mask:
        mask_shape = ((B * Co) // 128, 128)
        mask_spec = pl.BlockSpec(((tb * Co) // 128, 128), lambda bi: (bi, 0))
    else:
        mask_shape = (B, Co)
        mask_spec = pl.BlockSpec((tb, Co), lambda bi: (bi, 0))
    if flat_norm:
        norm_shape = (B,)
        norm_spec = pl.BlockSpec((B,), lambda bi: (0,))
        scratch_shapes = [pltpu.VMEM((B, Co), jnp.float32)]
    else:
        norm_shape = (B, 1)
        norm_spec = pl.BlockSpec((tb, 1), lambda bi: (bi, 0))
        scratch_shapes = []
    return pl.pallas_call(
        body,
        out_shape=(jax.ShapeDtypeStruct(mask_shape, jnp.float32),
                   jax.ShapeDtypeStruct(norm_shape, jnp.float32)),
        grid_spec=pltpu.PrefetchScalarGridSpec(
            num_scalar_prefetch=0,
            grid=(nb,),
            in_specs=[pl.BlockSpec((tb, S, C), lambda bi: (bi, 0, 0)),
                      pl.BlockSpec((Cb, C), rep),
                      pl.BlockSpec((Cb, Co), rep),
                      pl.BlockSpec((1, Cb), rep),
                      pl.BlockSpec((1, Cb), rep),
                      pl.BlockSpec((1, Cb), rep),
                      pl.BlockSpec((1, Cb), rep)],
            out_specs=(mask_spec, norm_spec),
            scratch_shapes=scratch_shapes,
        ),
        compiler_params=pltpu.CompilerParams(
            dimension_semantics=("arbitrary",),
            vmem_limit_bytes=56 * 1024 * 1024),
    )(xsc, w1t, w2f, gamma, beta, mean, var)


def kernel(x, w1, bn_gamma, bn_beta, bn_mean, bn_var, w2):
    B, C, H, W = x.shape
    S = H * W
    Cb = w1.shape[1]
    Co = w2.shape[1]

    xsc = jnp.transpose(x, (0, 2, 3, 1)).reshape(B, S, C)
    w1t = jnp.transpose(w1)

    tb = B
    for cand in (32, 16, 8):
        if B % cand == 0 and cand * S * C * x.dtype.itemsize <= 8 * 1024 * 1024:
            tb = cand
            break
    while tb > 1 and tb * S * C * x.dtype.itemsize > 16 * 1024 * 1024 \
            and tb % 2 == 0:
        tb //= 2

    f32 = jnp.float32
    mask2d, norm2d = _fused_call(
        xsc, w1t.astype(f32), w2.astype(f32),
        bn_gamma.astype(f32).reshape(1, Cb),
        bn_beta.astype(f32).reshape(1, Cb),
        bn_mean.astype(f32).reshape(1, Cb),
        bn_var.astype(f32).reshape(1, Cb),
        tb)

    mask_c = mask2d.reshape(B, Co, 1, 1)
    norm = norm2d.reshape(B)
    norm_t = jnp.array([Co], dtype=f32)
    return mask_c, norm, norm_t
